# baseline (speedup 1.0000x reference)
import functools

import jax
import jax.numpy as jnp
from jax import lax
from jax.experimental import pallas as pl
from jax.experimental.pallas import tpu as pltpu
from jax.experimental.pallas import tpu_sc as plsc

_ROWS = 4096
_CHUNK = 8
_SC_ROWS = 16384
_NW = 32
_PER_W = _SC_ROWS * 128 // _NW
_CE = 8192
_NCH = _PER_W // _CE

_LN2 = 0.6931471805599453


def _tc_body(mask_ref, o_fn_ref, g_fn_ref, m_fn_ref, o_fp_ref, g_fp_ref,
             m_fp_ref, res_ref, acc_ref):
    step = pl.program_id(0)

    @pl.when(step == 0)
    def _init():
        for i in range(4):
            acc_ref[i] = 0.0

    def chunk(j, carry):
        tot_fn, neg_fn, tot_fp, neg_fp = carry
        r = j * _CHUNK

        def branch(o_ref, g_ref, m_ref):
            o = o_ref[pl.ds(r, _CHUNK), :]
            g = g_ref[pl.ds(r, _CHUNK), :]
            m = m_ref[pl.ds(r, _CHUNK), :]
            om = 1.0 - o
            g1 = 1.0 - g
            g2 = g1 * g1
            g4 = g2 * g2
            isn = m == 0
            x = jnp.where(isn, om, o)
            xm = jnp.where(isn, o, om)
            w = xm * xm * jnp.where(isn, g4, 1.0)
            t = jnp.log(x) * w
            tn = jnp.where(isn, t, 0.0)
            return t, tn

        t1, tn1 = branch(o_fn_ref, g_fn_ref, m_fn_ref)
        t2, tn2 = branch(o_fp_ref, g_fp_ref, m_fp_ref)
        return (tot_fn + t1, neg_fn + tn1, tot_fp + t2, neg_fp + tn2)

    z = jnp.zeros((_CHUNK, 128), jnp.float32)
    tot_fn, neg_fn, tot_fp, neg_fp = lax.fori_loop(
        0, _ROWS // _CHUNK, chunk, (z, z, z, z), unroll=8)

    acc_ref[0] = acc_ref[0] + jnp.sum(neg_fn)
    acc_ref[1] = acc_ref[1] + jnp.sum(tot_fn)
    acc_ref[2] = acc_ref[2] + jnp.sum(neg_fp)
    acc_ref[3] = acc_ref[3] + jnp.sum(tot_fp)

    @pl.when(step == pl.num_programs(0) - 1)
    def _fini():
        for i in range(4):
            res_ref[i] = acc_ref[i]
        res_ref[4] = jnp.sum(mask_ref[...])
        res_ref[5] = 0.0
        res_ref[6] = 0.0
        res_ref[7] = 0.0


def _softlog(x):
    xi = lax.bitcast_convert_type(x, jnp.int32)
    e = (xi >> 23) - 127
    mi = (xi & jnp.int32(0x007FFFFF)) | jnp.int32(0x3F800000)
    mf = lax.bitcast_convert_type(mi, jnp.float32)
    zz = mf - 1.0
    ss = zz / (zz + 2.0)
    s2 = ss * ss
    p = ((((2.0 / 9.0) * s2 + (2.0 / 7.0)) * s2 + (2.0 / 5.0)) * s2
         + (2.0 / 3.0)) * s2 + 2.0
    return e.astype(jnp.float32) * _LN2 + p * ss


def _sc_kernel_fn(o_fn, g_fn, m_fn, o_fp, g_fp, m_fp, out_hbm,
                  b0, b1, b2, b3, b4, b5, c0, c1, c2, c3, c4, c5,
                  sem_a, sem_b, stage):
    wid = lax.axis_index("s") * 2 + lax.axis_index("c")
    base = wid * _PER_W

    hbms = (o_fn, g_fn, m_fn, o_fp, g_fp, m_fp)
    slots = ((b0, b1, b2, b3, b4, b5), (c0, c1, c2, c3, c4, c5))
    sems = (sem_a, sem_b)

    def issue(k, slot):
        off = base + k * _CE
        return [pltpu.async_copy(h.at[pl.ds(off, _CE)], buf, sems[slot])
                for h, buf in zip(hbms, slots[slot])]

    def compute(slot, accs):
        ob, gb, mb, ob2, gb2, mb2 = slots[slot]

        def inner(i, acc):
            a0, a1, a2, a3 = acc
            sl = pl.ds(i * 16, 16)

            def branch(obuf, gbuf, mbuf):
                o = obuf[sl]
                g = gbuf[sl]
                m = mbuf[sl]
                om = 1.0 - o
                g1 = 1.0 - g
                g2 = g1 * g1
                g4 = g2 * g2
                isn = m == 0
                x = jnp.where(isn, om, o)
                xm = jnp.where(isn, o, om)
                w = xm * xm * jnp.where(isn, g4, 1.0)
                t = _softlog(x) * w
                tn = jnp.where(isn, t, 0.0)
                return t, tn

            t1, tn1 = branch(ob, gb, mb)
            t2, tn2 = branch(ob2, gb2, mb2)
            return (a0 + tn1, a1 + t1, a2 + tn2, a3 + t2)

        return lax.fori_loop(0, _CE // 16, inner, accs, unroll=8)

    z16 = jnp.zeros((16,), jnp.float32)
    accs = (z16, z16, z16, z16)
    pending = issue(0, 0)
    for k in range(_NCH):
        slot = k % 2
        nxt = issue(k + 1, 1 - slot) if k + 1 < _NCH else None
        for h in pending:
            h.wait()
        accs = compute(slot, accs)
        pending = nxt

    stage[pl.ds(0, 16)] = accs[0]
    stage[pl.ds(16, 16)] = accs[1]
    stage[pl.ds(32, 16)] = accs[2]
    stage[pl.ds(48, 16)] = accs[3]
    pltpu.sync_copy(stage, out_hbm.at[wid])


def _sc_partials(o_fn, g_fn, m_fn, o_fp, g_fp, m_fp):
    mesh = plsc.VectorSubcoreMesh(core_axis_name="c", subcore_axis_name="s")
    f = functools.partial(
        pl.kernel,
        out_type=jax.ShapeDtypeStruct((_NW, 64), jnp.float32),
        mesh=mesh,
        scratch_types=(
            [pltpu.VMEM((_CE,), jnp.float32) for _ in range(2)]
            + [pltpu.VMEM((_CE,), jnp.int32)]
            + [pltpu.VMEM((_CE,), jnp.float32) for _ in range(2)]
            + [pltpu.VMEM((_CE,), jnp.int32)]
            + [pltpu.VMEM((_CE,), jnp.float32) for _ in range(2)]
            + [pltpu.VMEM((_CE,), jnp.int32)]
            + [pltpu.VMEM((_CE,), jnp.float32) for _ in range(2)]
            + [pltpu.VMEM((_CE,), jnp.int32)]
            + [pltpu.SemaphoreType.DMA, pltpu.SemaphoreType.DMA]
            + [pltpu.VMEM((64,), jnp.float32)]
        ),
    )(_sc_kernel_fn)
    return f(o_fn, g_fn, m_fn, o_fp, g_fp, m_fp)


def kernel(out, out_resi, target_resi, mask, negloss_fn_gt, fn_mask,
           negloss_fp_gt, fp_mask, wh_):
    B, C, H, W = out.shape
    R = B * C * H
    view = lambda a: a.reshape(R, W)
    flat = lambda a: a.reshape(R * W)

    sc_out = _sc_partials(flat(out), flat(negloss_fn_gt), flat(fn_mask),
                          flat(out_resi), flat(negloss_fp_gt), flat(fp_mask))
    scs = sc_out.reshape(_NW, 4, 16).sum(axis=(0, 2))

    grid = ((R - _SC_ROWS) // _ROWS,)
    off = _SC_ROWS // _ROWS
    big = pl.BlockSpec((_ROWS, W), lambda i: (i + off, 0))
    tc = pl.pallas_call(
        _tc_body,
        grid=grid,
        in_specs=[
            pl.BlockSpec(mask.shape, lambda i: (0, 0)),
            big, big, big, big, big, big,
        ],
        out_specs=pl.BlockSpec(memory_space=pltpu.SMEM),
        out_shape=jax.ShapeDtypeStruct((8,), jnp.float32),
        scratch_shapes=[pltpu.SMEM((4,), jnp.float32)],
    )(mask, view(out), view(negloss_fn_gt), view(fn_mask),
      view(out_resi), view(negloss_fp_gt), view(fp_mask))

    neg_fn = tc[0] + scs[0]
    tot_fn = tc[1] + scs[1]
    neg_fp = tc[2] + scs[2]
    tot_fp = tc[3] + scs[3]
    num_pos = tc[4]
    loss_fn = jnp.where(num_pos == 0.0, -neg_fn, -tot_fn)
    loss_fp = jnp.where(num_pos == 0.0, -neg_fp, -tot_fp)
    return loss_fn, loss_fp


# speedup vs baseline: 1.0179x; 1.0179x over previous
import functools

import jax
import jax.numpy as jnp
from jax import lax
from jax.experimental import pallas as pl
from jax.experimental.pallas import tpu as pltpu
from jax.experimental.pallas import tpu_sc as plsc

_ROWS = 4096
_CHUNK = 8
_SC_ROWS = 12288
_NW = 32
_PER_W = _SC_ROWS * 128 // _NW
_CE = 8192
_NCH = _PER_W // _CE

_LN2 = 0.6931471805599453


def _tc_body(mask_ref, o_fn_ref, g_fn_ref, m_fn_ref, o_fp_ref, g_fp_ref,
             m_fp_ref, res_ref, acc_ref):
    step = pl.program_id(0)

    @pl.when(step == 0)
    def _init():
        for i in range(4):
            acc_ref[i] = 0.0

    def chunk(j, carry):
        tot_fn, neg_fn, tot_fp, neg_fp = carry
        r = j * _CHUNK

        def branch(o_ref, g_ref, m_ref):
            o = o_ref[pl.ds(r, _CHUNK), :]
            g = g_ref[pl.ds(r, _CHUNK), :]
            m = m_ref[pl.ds(r, _CHUNK), :]
            om = 1.0 - o
            g1 = 1.0 - g
            g2 = g1 * g1
            g4 = g2 * g2
            isn = m == 0
            x = jnp.where(isn, om, o)
            xm = jnp.where(isn, o, om)
            w = xm * xm * jnp.where(isn, g4, 1.0)
            t = jnp.log(x) * w
            tn = jnp.where(isn, t, 0.0)
            return t, tn

        t1, tn1 = branch(o_fn_ref, g_fn_ref, m_fn_ref)
        t2, tn2 = branch(o_fp_ref, g_fp_ref, m_fp_ref)
        return (tot_fn + t1, neg_fn + tn1, tot_fp + t2, neg_fp + tn2)

    z = jnp.zeros((_CHUNK, 128), jnp.float32)
    tot_fn, neg_fn, tot_fp, neg_fp = lax.fori_loop(
        0, _ROWS // _CHUNK, chunk, (z, z, z, z), unroll=8)

    acc_ref[0] = acc_ref[0] + jnp.sum(neg_fn)
    acc_ref[1] = acc_ref[1] + jnp.sum(tot_fn)
    acc_ref[2] = acc_ref[2] + jnp.sum(neg_fp)
    acc_ref[3] = acc_ref[3] + jnp.sum(tot_fp)

    @pl.when(step == pl.num_programs(0) - 1)
    def _fini():
        for i in range(4):
            res_ref[i] = acc_ref[i]
        res_ref[4] = jnp.sum(mask_ref[...])
        res_ref[5] = 0.0
        res_ref[6] = 0.0
        res_ref[7] = 0.0


def _softlog(x):
    xi = lax.bitcast_convert_type(x, jnp.int32)
    e = (xi >> 23) - 127
    mi = (xi & jnp.int32(0x007FFFFF)) | jnp.int32(0x3F800000)
    mf = lax.bitcast_convert_type(mi, jnp.float32)
    zz = mf - 1.0
    ss = zz / (zz + 2.0)
    s2 = ss * ss
    p = ((((2.0 / 9.0) * s2 + (2.0 / 7.0)) * s2 + (2.0 / 5.0)) * s2
         + (2.0 / 3.0)) * s2 + 2.0
    return e.astype(jnp.float32) * _LN2 + p * ss


def _sc_kernel_fn(o_fn, g_fn, m_fn, o_fp, g_fp, m_fp, out_hbm,
                  b0, b1, b2, b3, b4, b5, c0, c1, c2, c3, c4, c5,
                  sem_a, sem_b, stage):
    wid = lax.axis_index("s") * 2 + lax.axis_index("c")
    base = wid * _PER_W

    hbms = (o_fn, g_fn, m_fn, o_fp, g_fp, m_fp)
    slots = ((b0, b1, b2, b3, b4, b5), (c0, c1, c2, c3, c4, c5))
    sems = (sem_a, sem_b)

    def issue(k, slot):
        off = base + k * _CE
        return [pltpu.async_copy(h.at[pl.ds(off, _CE)], buf, sems[slot])
                for h, buf in zip(hbms, slots[slot])]

    def compute(slot, accs):
        ob, gb, mb, ob2, gb2, mb2 = slots[slot]

        def inner(i, acc):
            a0, a1, a2, a3 = acc
            sl = pl.ds(i * 16, 16)

            def branch(obuf, gbuf, mbuf):
                o = obuf[sl]
                g = gbuf[sl]
                m = mbuf[sl]
                om = 1.0 - o
                g1 = 1.0 - g
                g2 = g1 * g1
                g4 = g2 * g2
                isn = m == 0
                x = jnp.where(isn, om, o)
                xm = jnp.where(isn, o, om)
                w = xm * xm * jnp.where(isn, g4, 1.0)
                t = _softlog(x) * w
                tn = jnp.where(isn, t, 0.0)
                return t, tn

            t1, tn1 = branch(ob, gb, mb)
            t2, tn2 = branch(ob2, gb2, mb2)
            return (a0 + tn1, a1 + t1, a2 + tn2, a3 + t2)

        return lax.fori_loop(0, _CE // 16, inner, accs, unroll=4)

    z16 = jnp.zeros((16,), jnp.float32)
    accs = (z16, z16, z16, z16)
    pending = issue(0, 0)
    for k in range(_NCH):
        slot = k % 2
        nxt = issue(k + 1, 1 - slot) if k + 1 < _NCH else None
        for h in pending:
            h.wait()
        accs = compute(slot, accs)
        pending = nxt

    stage[pl.ds(0, 16)] = accs[0]
    stage[pl.ds(16, 16)] = accs[1]
    stage[pl.ds(32, 16)] = accs[2]
    stage[pl.ds(48, 16)] = accs[3]
    pltpu.sync_copy(stage, out_hbm.at[wid])


def _sc_partials(o_fn, g_fn, m_fn, o_fp, g_fp, m_fp):
    mesh = plsc.VectorSubcoreMesh(core_axis_name="c", subcore_axis_name="s")
    f = functools.partial(
        pl.kernel,
        out_type=jax.ShapeDtypeStruct((_NW, 64), jnp.float32),
        mesh=mesh,
        scratch_types=(
            [pltpu.VMEM((_CE,), jnp.float32) for _ in range(2)]
            + [pltpu.VMEM((_CE,), jnp.int32)]
            + [pltpu.VMEM((_CE,), jnp.float32) for _ in range(2)]
            + [pltpu.VMEM((_CE,), jnp.int32)]
            + [pltpu.VMEM((_CE,), jnp.float32) for _ in range(2)]
            + [pltpu.VMEM((_CE,), jnp.int32)]
            + [pltpu.VMEM((_CE,), jnp.float32) for _ in range(2)]
            + [pltpu.VMEM((_CE,), jnp.int32)]
            + [pltpu.SemaphoreType.DMA, pltpu.SemaphoreType.DMA]
            + [pltpu.VMEM((64,), jnp.float32)]
        ),
    )(_sc_kernel_fn)
    return f(o_fn, g_fn, m_fn, o_fp, g_fp, m_fp)


def kernel(out, out_resi, target_resi, mask, negloss_fn_gt, fn_mask,
           negloss_fp_gt, fp_mask, wh_):
    B, C, H, W = out.shape
    R = B * C * H
    view = lambda a: a.reshape(R, W)
    flat = lambda a: a.reshape(R * W)

    sc_out = _sc_partials(flat(out), flat(negloss_fn_gt), flat(fn_mask),
                          flat(out_resi), flat(negloss_fp_gt), flat(fp_mask))
    scs = sc_out.reshape(_NW, 4, 16).sum(axis=(0, 2))

    grid = ((R - _SC_ROWS) // _ROWS,)
    off = _SC_ROWS // _ROWS
    big = pl.BlockSpec((_ROWS, W), lambda i: (i + off, 0))
    tc = pl.pallas_call(
        _tc_body,
        grid=grid,
        in_specs=[
            pl.BlockSpec(mask.shape, lambda i: (0, 0)),
            big, big, big, big, big, big,
        ],
        out_specs=pl.BlockSpec(memory_space=pltpu.SMEM),
        out_shape=jax.ShapeDtypeStruct((8,), jnp.float32),
        scratch_shapes=[pltpu.SMEM((4,), jnp.float32)],
    )(mask, view(out), view(negloss_fn_gt), view(fn_mask),
      view(out_resi), view(negloss_fp_gt), view(fp_mask))

    neg_fn = tc[0] + scs[0]
    tot_fn = tc[1] + scs[1]
    neg_fp = tc[2] + scs[2]
    tot_fp = tc[3] + scs[3]
    num_pos = tc[4]
    loss_fn = jnp.where(num_pos == 0.0, -neg_fn, -tot_fn)
    loss_fp = jnp.where(num_pos == 0.0, -neg_fp, -tot_fp)
    return loss_fn, loss_fp


# speedup vs baseline: 1.3304x; 1.3070x over previous
import jax
import jax.numpy as jnp
from jax.experimental import pallas as pl
from jax.experimental.pallas import tpu as pltpu

_ROWS = 5120
_CHUNK = 8


def _body(mask_ref, o_fn_ref, g_fn_ref, m_fn_ref, o_fp_ref, g_fp_ref,
          m_fp_ref, res_ref, acc_ref):
    step = pl.program_id(0)

    @pl.when(step == 0)
    def _init():
        for i in range(4):
            acc_ref[i] = 0.0

    def chunk(j, carry):
        tot_fn, neg_fn, tot_fp, neg_fp = carry
        r = j * _CHUNK

        def branch(o_ref, g_ref, m_ref):
            o = o_ref[pl.ds(r, _CHUNK), :]
            g = g_ref[pl.ds(r, _CHUNK), :]
            m = m_ref[pl.ds(r, _CHUNK), :]
            om = 1.0 - o
            g1 = 1.0 - g
            g2 = g1 * g1
            g4 = g2 * g2
            isn = m == 0
            x = jnp.where(isn, om, o)
            xm = jnp.where(isn, o, om)
            w = xm * xm * jnp.where(isn, g4, 1.0)
            t = jnp.log(x) * w
            tn = jnp.where(isn, t, 0.0)
            return t, tn

        t1, tn1 = branch(o_fn_ref, g_fn_ref, m_fn_ref)
        t2, tn2 = branch(o_fp_ref, g_fp_ref, m_fp_ref)
        return (tot_fn + t1, neg_fn + tn1, tot_fp + t2, neg_fp + tn2)

    z = jnp.zeros((_CHUNK, 128), jnp.float32)
    tot_fn, neg_fn, tot_fp, neg_fp = jax.lax.fori_loop(
        0, _ROWS // _CHUNK, chunk, (z, z, z, z), unroll=8)

    acc_ref[0] = acc_ref[0] + jnp.sum(neg_fn)
    acc_ref[1] = acc_ref[1] + jnp.sum(tot_fn)
    acc_ref[2] = acc_ref[2] + jnp.sum(neg_fp)
    acc_ref[3] = acc_ref[3] + jnp.sum(tot_fp)

    @pl.when(step == pl.num_programs(0) - 1)
    def _fini():
        num_pos = jnp.sum(mask_ref[...])
        res_ref[0] = jnp.where(num_pos == 0.0, -acc_ref[0], -acc_ref[1])
        res_ref[1] = jnp.where(num_pos == 0.0, -acc_ref[2], -acc_ref[3])


def kernel(out, out_resi, target_resi, mask, negloss_fn_gt, fn_mask,
           negloss_fp_gt, fp_mask, wh_):
    B, C, H, W = out.shape
    R = B * C * H
    view = lambda a: a.reshape(R, W)
    grid = (R // _ROWS,)
    big = pl.BlockSpec((_ROWS, W), lambda i: (i, 0))
    res = pl.pallas_call(
        _body,
        grid=grid,
        in_specs=[
            pl.BlockSpec(mask.shape, lambda i: (0, 0)),
            big, big, big, big, big, big,
        ],
        out_specs=pl.BlockSpec(memory_space=pltpu.SMEM),
        out_shape=jax.ShapeDtypeStruct((2,), jnp.float32),
        scratch_shapes=[pltpu.SMEM((4,), jnp.float32)],
    )(mask, view(out), view(negloss_fn_gt), view(fn_mask),
      view(out_resi), view(negloss_fp_gt), view(fp_mask))
    return res[0], res[1]
